# baseline (speedup 1.0000x reference)
import functools

import jax
import jax.numpy as jnp
from jax import lax
from jax.experimental import pallas as pl
from jax.experimental.pallas import tpu as pltpu
from jax.experimental.pallas import tpu_sc as plsc

B = 16384
ED = 64
NC, NS = 2, 16
NW = NC * NS
BPW = B // NW

_sc_mesh = plsc.VectorSubcoreMesh(core_axis_name="c", subcore_axis_name="s")


@functools.partial(
    pl.kernel,
    mesh=_sc_mesh,
    out_type=[
        jax.ShapeDtypeStruct((B, 2 * ED), jnp.float32),
        jax.ShapeDtypeStruct((B, 2 * ED), jnp.float32),
    ],
    scratch_types=[
        pltpu.VMEM((BPW,), jnp.int32),
        pltpu.VMEM((BPW,), jnp.int32),
        pltpu.VMEM((BPW, 2 * ED), jnp.float32),
        pltpu.SemaphoreType.DMA,
    ],
)
def _gather_sc(uid_hbm, iid_hbm, utab_hbm, itab_hbm, uout_hbm, iout_hbm,
               uidx_v, iidx_v, rows_v, sem):
    wid = lax.axis_index("s") * NC + lax.axis_index("c")
    base = wid * BPW
    pltpu.sync_copy(uid_hbm.at[pl.ds(base, BPW)], uidx_v)
    pltpu.sync_copy(iid_hbm.at[pl.ds(base, BPW)], iidx_v)
    pltpu.async_copy(utab_hbm.at[uidx_v], rows_v, sem).wait()
    pltpu.sync_copy(rows_v, uout_hbm.at[pl.ds(base, BPW)])
    pltpu.async_copy(itab_hbm.at[iidx_v], rows_v, sem).wait()
    pltpu.sync_copy(rows_v, iout_hbm.at[pl.ds(base, BPW)])


BLK = 2048


def _mlp_body(feat_ref, up_ref, ip_ref, uid_ref, iid_ref,
              w1_ref, b1_ref, w2_ref, b2_ref,
              w3u_ref, w3i_ref, w3f_ref, b3_ref, w4_ref, b4_ref,
              w5_ref, b5_ref, out_ref):
    up = up_ref[...]
    ip = ip_ref[...]
    u_odd = (uid_ref[...] & 1) == 1
    i_odd = (iid_ref[...] & 1) == 1
    ue = jnp.where(u_odd, up[:, ED:], up[:, :ED])
    ie = jnp.where(i_odd, ip[:, ED:], ip[:, :ED])
    h = jnp.maximum(
        jnp.dot(feat_ref[...], w1_ref[...], preferred_element_type=jnp.float32)
        + b1_ref[...], 0.0)
    f = jnp.maximum(
        jnp.dot(h, w2_ref[...], preferred_element_type=jnp.float32)
        + b2_ref[...], 0.0)
    y = (jnp.dot(ue, w3u_ref[...], preferred_element_type=jnp.float32)
         + jnp.dot(ie, w3i_ref[...], preferred_element_type=jnp.float32)
         + jnp.dot(f, w3f_ref[...], preferred_element_type=jnp.float32)
         + b3_ref[...])
    y = jnp.maximum(y, 0.0)
    y = jnp.maximum(
        jnp.dot(y, w4_ref[...], preferred_element_type=jnp.float32)
        + b4_ref[...], 0.0)
    z = (jnp.dot(y, w5_ref[...], preferred_element_type=jnp.float32)
         + b5_ref[...])
    out_ref[...] = 1.0 / (1.0 + jnp.exp(-z))


def _mlp_tc(features, up, ip, uid, iid,
            W1, b1, W2, b2, W3u, W3i, W3f, b3, W4, b4, W5, b5):
    nblk = B // BLK
    row_spec = lambda w: pl.BlockSpec((BLK, w), lambda i: (i, 0))
    full = lambda a: pl.BlockSpec(a.shape, lambda i: (0,) * a.ndim)
    return pl.pallas_call(
        _mlp_body,
        grid=(nblk,),
        in_specs=[
            row_spec(features.shape[1]),
            row_spec(2 * ED),
            row_spec(2 * ED),
            row_spec(1),
            row_spec(1),
            full(W1), full(b1), full(W2), full(b2),
            full(W3u), full(W3i), full(W3f), full(b3),
            full(W4), full(b4), full(W5), full(b5),
        ],
        out_specs=pl.BlockSpec((BLK, 1), lambda i: (i, 0)),
        out_shape=jax.ShapeDtypeStruct((B, 1), jnp.float32),
    )(features, up, ip, uid, iid,
      W1, b1, W2, b2, W3u, W3i, W3f, b3, W4, b4, W5, b5)


def kernel(user_ids, item_ids, features, user_table, item_table,
           W1, b1, W2, b2, W3, b3, W4, b4, W5, b5):
    uid = user_ids.astype(jnp.int32)
    iid = item_ids.astype(jnp.int32)
    up, ip = _gather_sc(
        lax.shift_right_logical(uid, 1), lax.shift_right_logical(iid, 1),
        user_table.reshape(-1, 2 * ED), item_table.reshape(-1, 2 * ED))
    out = _mlp_tc(
        features, up, ip, uid.reshape(B, 1), iid.reshape(B, 1),
        W1, b1.reshape(1, -1), W2, b2.reshape(1, -1),
        W3[:ED], W3[ED:2 * ED], W3[2 * ED:], b3.reshape(1, -1),
        W4, b4.reshape(1, -1), W5, b5.reshape(1, -1))
    return out.reshape(B)


# speedup vs baseline: 1.6371x; 1.6371x over previous
import functools

import jax
import jax.numpy as jnp
from jax import lax
from jax.experimental import pallas as pl
from jax.experimental.pallas import tpu as pltpu
from jax.experimental.pallas import tpu_sc as plsc

B = 16384
BSC = 8192
BTC = B - BSC
ED = 64
LANES = 128
NC, NS = 2, 16
NW = NC * NS
BPW = BSC // NW
NBUF = 4
CHUNK = 16
HALF = 128
S = 16

_sc_mesh = plsc.VectorSubcoreMesh(core_axis_name="c", subcore_axis_name="s")


@functools.partial(
    pl.kernel,
    mesh=_sc_mesh,
    out_type=[
        jax.ShapeDtypeStruct((ED, BSC), jnp.float32),
        jax.ShapeDtypeStruct((ED, BSC), jnp.float32),
    ],
    scratch_types=[
        pltpu.VMEM((BPW,), jnp.int32),
        pltpu.VMEM((BPW,), jnp.int32),
        pltpu.VMEM((NBUF, ED, LANES), jnp.float32),
        pltpu.VMEM((NBUF, ED, LANES), jnp.float32),
        pltpu.VMEM((ED, HALF), jnp.float32),
        pltpu.VMEM((ED, HALF), jnp.float32),
        pltpu.SemaphoreType.DMA((NBUF,)),
        pltpu.SemaphoreType.DMA((NBUF,)),
    ],
    compiler_params=pltpu.CompilerParams(needs_layout_passes=False),
)
def _gather_sc(uid_hbm, iid_hbm, utabT_hbm, itabT_hbm, uoutT_hbm, ioutT_hbm,
               uid_v, iid_v, uslab, islab, uout_v, iout_v, usem, isem):
    wid = lax.axis_index("s") * NC + lax.axis_index("c")
    base = wid * BPW
    pltpu.sync_copy(uid_hbm.at[pl.ds(base, BPW)], uid_v)
    pltpu.sync_copy(iid_hbm.at[pl.ds(base, BPW)], iid_v)

    rows16 = lax.iota(jnp.int32, 16)

    def fire(u, v, b):
        ut = pl.multiple_of((u >> 7) * LANES, LANES)
        pltpu.async_copy(utabT_hbm.at[:, pl.ds(ut, LANES)], uslab.at[b],
                         usem.at[b])
        it = pl.multiple_of((v >> 7) * LANES, LANES)
        pltpu.async_copy(itabT_hbm.at[:, pl.ds(it, LANES)], islab.at[b],
                         isem.at[b])

    def extract(u, v, col_i, b):
        pltpu.make_async_copy(utabT_hbm.at[:, pl.ds(0, LANES)], uslab.at[b],
                              usem.at[b]).wait()
        pltpu.make_async_copy(itabT_hbm.at[:, pl.ds(0, LANES)], islab.at[b],
                              isem.at[b]).wait()
        ul = jnp.full((16,), u & (LANES - 1), jnp.int32)
        il = jnp.full((16,), v & (LANES - 1), jnp.int32)
        col = jnp.full((16,), col_i, jnp.int32)
        for c in range(ED // 16):
            r = rows16 + (16 * c)
            uvec = plsc.load_gather(uslab.at[b], [r, ul])
            plsc.store_scatter(uout_v, [r, col], uvec)
            ivec = plsc.load_gather(islab.at[b], [r, il])
            plsc.store_scatter(iout_v, [r, col], ivec)

    for h in range(BPW // HALF):
        h0 = h * HALF

        @pl.loop(h0, h0 + HALF, step=CHUNK)
        def _chunk(o):
            uvec = uid_v[pl.ds(o, CHUNK)]
            ivec = iid_v[pl.ds(o, CHUNK)]

            for j in range(CHUNK):
                i = o + j
                b = j % NBUF
                pj = (j - NBUF) % CHUNK

                @pl.when(i - h0 >= NBUF)
                def _():
                    po = o if j >= NBUF else o - CHUNK
                    puvec = uid_v[pl.ds(po, CHUNK)]
                    pivec = iid_v[pl.ds(po, CHUNK)]
                    extract(puvec[pj], pivec[pj], (po + pj) - h0, b)

                fire(uvec[j], ivec[j], b)

        last = h0 + HALF - CHUNK
        luvec = uid_v[pl.ds(last, CHUNK)]
        livec = iid_v[pl.ds(last, CHUNK)]
        for j in range(NBUF):
            pj = CHUNK - NBUF + j
            extract(luvec[pj], livec[pj], HALF - NBUF + j, pj % NBUF)

        pltpu.sync_copy(uout_v, uoutT_hbm.at[:, pl.ds(base + h0, HALF)])
        pltpu.sync_copy(iout_v, ioutT_hbm.at[:, pl.ds(base + h0, HALF)])


def _tc_gather_body(ids_ref, *refs):
    uslabs = refs[:S]
    islabs = refs[S:2 * S]
    uout_ref, iout_ref = refs[2 * S], refs[2 * S + 1]
    i = pl.program_id(0)
    iota = lax.broadcasted_iota(jnp.int32, (LANES, 1), 0)
    for k in range(S):
        ul = ids_ref[2 * (i * S + k)] & (LANES - 1)
        il = ids_ref[2 * (i * S + k) + 1] & (LANES - 1)
        uoh = (iota == ul).astype(jnp.float32)
        ioh = (iota == il).astype(jnp.float32)
        uout_ref[0, :, k:k + 1] = jnp.dot(uslabs[k][...], uoh,
                                          preferred_element_type=jnp.float32)
        iout_ref[0, :, k:k + 1] = jnp.dot(islabs[k][...], ioh,
                                          preferred_element_type=jnp.float32)


def _tc_gather(ids2, utabT, itabT):
    nsteps = BTC // S

    def uspec(k):
        return pl.BlockSpec(
            (ED, LANES), lambda i, ids: (0, ids[2 * (i * S + k)] >> 7))

    def ispec(k):
        return pl.BlockSpec(
            (ED, LANES), lambda i, ids: (0, ids[2 * (i * S + k) + 1] >> 7))

    grid_spec = pltpu.PrefetchScalarGridSpec(
        num_scalar_prefetch=1,
        grid=(nsteps,),
        in_specs=([uspec(k) for k in range(S)]
                  + [ispec(k) for k in range(S)]),
        out_specs=[
            pl.BlockSpec((1, ED, S), lambda i, ids: (i, 0, 0)),
            pl.BlockSpec((1, ED, S), lambda i, ids: (i, 0, 0)),
        ],
    )
    u3, i3 = pl.pallas_call(
        _tc_gather_body,
        grid_spec=grid_spec,
        out_shape=[
            jax.ShapeDtypeStruct((nsteps, ED, S), jnp.float32),
            jax.ShapeDtypeStruct((nsteps, ED, S), jnp.float32),
        ],
    )(ids2, *([utabT] * S), *([itabT] * S))
    ueT = u3.transpose(1, 0, 2).reshape(ED, BTC)
    ieT = i3.transpose(1, 0, 2).reshape(ED, BTC)
    return ueT, ieT


BLK = 2048


def _mlp_body(featT_ref, ueT_ref, ieT_ref, w1t_ref, b1_ref, w2t_ref, b2_ref,
              w3ut_ref, w3it_ref, w3ft_ref, b3_ref, w4t_ref, b4_ref,
              w5t_ref, b5_ref, out_ref):
    dot = lambda a, b: jnp.dot(a, b, preferred_element_type=jnp.float32)
    h = jnp.maximum(dot(w1t_ref[...], featT_ref[...]) + b1_ref[...], 0.0)
    f = jnp.maximum(dot(w2t_ref[...], h) + b2_ref[...], 0.0)
    y = (dot(w3ut_ref[...], ueT_ref[...])
         + dot(w3it_ref[...], ieT_ref[...])
         + dot(w3ft_ref[...], f)
         + b3_ref[...])
    y = jnp.maximum(y, 0.0)
    y = jnp.maximum(dot(w4t_ref[...], y) + b4_ref[...], 0.0)
    z = dot(w5t_ref[...], y) + b5_ref[...]
    out_ref[...] = 1.0 / (1.0 + jnp.exp(-z))


def _mlp_tc(n, featT, ueT, ieT, W1T, b1, W2T, b2, W3uT, W3iT, W3fT, b3,
            W4T, b4, W5T, b5):
    nblk = n // BLK
    col_spec = lambda h: pl.BlockSpec((h, BLK), lambda i: (0, i))
    full = lambda a: pl.BlockSpec(a.shape, lambda i: (0,) * a.ndim)
    return pl.pallas_call(
        _mlp_body,
        grid=(nblk,),
        in_specs=[
            col_spec(featT.shape[0]),
            col_spec(ED),
            col_spec(ED),
            full(W1T), full(b1), full(W2T), full(b2),
            full(W3uT), full(W3iT), full(W3fT), full(b3),
            full(W4T), full(b4), full(W5T), full(b5),
        ],
        out_specs=pl.BlockSpec((1, BLK), lambda i: (0, i)),
        out_shape=jax.ShapeDtypeStruct((1, n), jnp.float32),
    )(featT, ueT, ieT, W1T, b1, W2T, b2, W3uT, W3iT, W3fT, b3,
      W4T, b4, W5T, b5)


def kernel(user_ids, item_ids, features, user_table, item_table,
           W1, b1, W2, b2, W3, b3, W4, b4, W5, b5):
    uid = user_ids.astype(jnp.int32)
    iid = item_ids.astype(jnp.int32)
    utabT = user_table.T
    itabT = item_table.T
    featT = features.T
    ueT_sc, ieT_sc = _gather_sc(uid, iid, utabT, itabT)
    ids2 = jnp.stack([uid[BSC:], iid[BSC:]], axis=1).reshape(-1)
    ueT_tc, ieT_tc = _tc_gather(ids2, utabT, itabT)
    weights = (W1.T, b1.reshape(-1, 1), W2.T, b2.reshape(-1, 1),
               W3[:ED].T, W3[ED:2 * ED].T, W3[2 * ED:].T, b3.reshape(-1, 1),
               W4.T, b4.reshape(-1, 1), W5.T, b5.reshape(-1, 1))
    out_a = _mlp_tc(BSC, featT[:, :BSC], ueT_sc, ieT_sc, *weights)
    out_b = _mlp_tc(BTC, featT[:, BSC:], ueT_tc, ieT_tc, *weights)
    return jnp.concatenate([out_a, out_b], axis=1).reshape(B)


# speedup vs baseline: 2.7126x; 1.6570x over previous
import functools

import jax
import jax.numpy as jnp
from jax import lax
from jax.experimental import pallas as pl
from jax.experimental.pallas import tpu as pltpu
from jax.experimental.pallas import tpu_sc as plsc

B = 16384
BSC = B
BTC = B - BSC
ED = 64
LANES = 128
NC, NS = 2, 16
NW = NC * NS
BPW = BSC // NW
NBUF = 4
CHUNK = 16
HALF = 128

_sc_mesh = plsc.VectorSubcoreMesh(core_axis_name="c", subcore_axis_name="s")


@functools.partial(
    pl.kernel,
    mesh=_sc_mesh,
    out_type=[
        jax.ShapeDtypeStruct((ED, BSC), jnp.float32),
        jax.ShapeDtypeStruct((ED, BSC), jnp.float32),
    ],
    scratch_types=[
        pltpu.VMEM((BPW,), jnp.int32),
        pltpu.VMEM((BPW,), jnp.int32),
        pltpu.VMEM((NBUF, ED, LANES), jnp.float32),
        pltpu.VMEM((NBUF, ED, LANES), jnp.float32),
        pltpu.VMEM((ED, HALF), jnp.float32),
        pltpu.VMEM((ED, HALF), jnp.float32),
        pltpu.SemaphoreType.DMA((NBUF,)),
        pltpu.SemaphoreType.DMA((NBUF,)),
    ],
    compiler_params=pltpu.CompilerParams(needs_layout_passes=False),
)
def _gather_sc(uid_hbm, iid_hbm, utabT_hbm, itabT_hbm, uoutT_hbm, ioutT_hbm,
               uid_v, iid_v, uslab, islab, uout_v, iout_v, usem, isem):
    wid = lax.axis_index("s") * NC + lax.axis_index("c")
    base = wid * BPW
    pltpu.sync_copy(uid_hbm.at[pl.ds(base, BPW)], uid_v)
    pltpu.sync_copy(iid_hbm.at[pl.ds(base, BPW)], iid_v)

    rows16 = lax.iota(jnp.int32, 16)

    def fire(u, v, b):
        ut = pl.multiple_of((u >> 7) * LANES, LANES)
        pltpu.async_copy(utabT_hbm.at[:, pl.ds(ut, LANES)], uslab.at[b],
                         usem.at[b])
        it = pl.multiple_of((v >> 7) * LANES, LANES)
        pltpu.async_copy(itabT_hbm.at[:, pl.ds(it, LANES)], islab.at[b],
                         isem.at[b])

    def extract(u, v, col_i, b):
        pltpu.make_async_copy(utabT_hbm.at[:, pl.ds(0, LANES)], uslab.at[b],
                              usem.at[b]).wait()
        pltpu.make_async_copy(itabT_hbm.at[:, pl.ds(0, LANES)], islab.at[b],
                              isem.at[b]).wait()
        ul = jnp.full((16,), u & (LANES - 1), jnp.int32)
        il = jnp.full((16,), v & (LANES - 1), jnp.int32)
        col = jnp.full((16,), col_i, jnp.int32)
        for c in range(ED // 16):
            r = rows16 + (16 * c)
            uvec = plsc.load_gather(uslab.at[b], [r, ul])
            plsc.store_scatter(uout_v, [r, col], uvec)
            ivec = plsc.load_gather(islab.at[b], [r, il])
            plsc.store_scatter(iout_v, [r, col], ivec)

    for h in range(BPW // HALF):
        h0 = h * HALF

        @pl.loop(h0, h0 + HALF, step=CHUNK)
        def _chunk(o):
            uvec = uid_v[pl.ds(o, CHUNK)]
            ivec = iid_v[pl.ds(o, CHUNK)]

            for j in range(CHUNK):
                i = o + j
                b = j % NBUF
                pj = (j - NBUF) % CHUNK

                @pl.when(i - h0 >= NBUF)
                def _():
                    po = o if j >= NBUF else o - CHUNK
                    puvec = uid_v[pl.ds(po, CHUNK)]
                    pivec = iid_v[pl.ds(po, CHUNK)]
                    extract(puvec[pj], pivec[pj], (po + pj) - h0, b)

                fire(uvec[j], ivec[j], b)

        last = h0 + HALF - CHUNK
        luvec = uid_v[pl.ds(last, CHUNK)]
        livec = iid_v[pl.ds(last, CHUNK)]
        for j in range(NBUF):
            pj = CHUNK - NBUF + j
            extract(luvec[pj], livec[pj], HALF - NBUF + j, pj % NBUF)

        pltpu.sync_copy(uout_v, uoutT_hbm.at[:, pl.ds(base + h0, HALF)])
        pltpu.sync_copy(iout_v, ioutT_hbm.at[:, pl.ds(base + h0, HALF)])


BLK = 2048


def _mlp_body(featT_ref, ueT_ref, ieT_ref, w1t_ref, b1_ref, w2t_ref, b2_ref,
              w3ut_ref, w3it_ref, w3ft_ref, b3_ref, w4t_ref, b4_ref,
              w5t_ref, b5_ref, out_ref):
    dot = lambda a, b: jnp.dot(a, b, preferred_element_type=jnp.float32)
    h = jnp.maximum(dot(w1t_ref[...], featT_ref[...]) + b1_ref[...], 0.0)
    f = jnp.maximum(dot(w2t_ref[...], h) + b2_ref[...], 0.0)
    y = (dot(w3ut_ref[...], ueT_ref[...])
         + dot(w3it_ref[...], ieT_ref[...])
         + dot(w3ft_ref[...], f)
         + b3_ref[...])
    y = jnp.maximum(y, 0.0)
    y = jnp.maximum(dot(w4t_ref[...], y) + b4_ref[...], 0.0)
    z = dot(w5t_ref[...], y) + b5_ref[...]
    out_ref[...] = 1.0 / (1.0 + jnp.exp(-z))


def _mlp_tc(n, featT, ueT, ieT, W1T, b1, W2T, b2, W3uT, W3iT, W3fT, b3,
            W4T, b4, W5T, b5):
    nblk = n // BLK
    col_spec = lambda h: pl.BlockSpec((h, BLK), lambda i: (0, i))
    full = lambda a: pl.BlockSpec(a.shape, lambda i: (0,) * a.ndim)
    return pl.pallas_call(
        _mlp_body,
        grid=(nblk,),
        in_specs=[
            col_spec(featT.shape[0]),
            col_spec(ED),
            col_spec(ED),
            full(W1T), full(b1), full(W2T), full(b2),
            full(W3uT), full(W3iT), full(W3fT), full(b3),
            full(W4T), full(b4), full(W5T), full(b5),
        ],
        out_specs=pl.BlockSpec((1, BLK), lambda i: (0, i)),
        out_shape=jax.ShapeDtypeStruct((1, n), jnp.float32),
    )(featT, ueT, ieT, W1T, b1, W2T, b2, W3uT, W3iT, W3fT, b3,
      W4T, b4, W5T, b5)


def kernel(user_ids, item_ids, features, user_table, item_table,
           W1, b1, W2, b2, W3, b3, W4, b4, W5, b5):
    uid = user_ids.astype(jnp.int32)
    iid = item_ids.astype(jnp.int32)
    utabT = user_table.T
    itabT = item_table.T
    featT = features.T
    ueT_sc, ieT_sc = _gather_sc(uid, iid, utabT, itabT)
    weights = (W1.T, b1.reshape(-1, 1), W2.T, b2.reshape(-1, 1),
               W3[:ED].T, W3[ED:2 * ED].T, W3[2 * ED:].T, b3.reshape(-1, 1),
               W4.T, b4.reshape(-1, 1), W5.T, b5.reshape(-1, 1))
    out = _mlp_tc(BSC, featT, ueT_sc, ieT_sc, *weights)
    return out.reshape(B)


# speedup vs baseline: 2.7466x; 1.0125x over previous
import functools

import jax
import jax.numpy as jnp
from jax import lax
from jax.experimental import pallas as pl
from jax.experimental.pallas import tpu as pltpu
from jax.experimental.pallas import tpu_sc as plsc

B = 16384
BSC = B
BTC = B - BSC
ED = 64
LANES = 128
NC, NS = 2, 16
NW = NC * NS
BPW = BSC // NW
NBUF = 4
CHUNK = 16
HALF = 256

_sc_mesh = plsc.VectorSubcoreMesh(core_axis_name="c", subcore_axis_name="s")


@functools.partial(
    pl.kernel,
    mesh=_sc_mesh,
    out_type=[
        jax.ShapeDtypeStruct((ED, BSC), jnp.float32),
        jax.ShapeDtypeStruct((ED, BSC), jnp.float32),
    ],
    scratch_types=[
        pltpu.VMEM((BPW,), jnp.int32),
        pltpu.VMEM((BPW,), jnp.int32),
        pltpu.VMEM((NBUF, ED, LANES), jnp.float32),
        pltpu.VMEM((NBUF, ED, LANES), jnp.float32),
        pltpu.VMEM((ED, HALF), jnp.float32),
        pltpu.VMEM((ED, HALF), jnp.float32),
        pltpu.SemaphoreType.DMA((NBUF,)),
        pltpu.SemaphoreType.DMA((NBUF,)),
    ],
    compiler_params=pltpu.CompilerParams(needs_layout_passes=False),
)
def _gather_sc(uid_hbm, iid_hbm, utabT_hbm, itabT_hbm, uoutT_hbm, ioutT_hbm,
               uid_v, iid_v, uslab, islab, uout_v, iout_v, usem, isem):
    wid = lax.axis_index("s") * NC + lax.axis_index("c")
    base = wid * BPW
    pltpu.sync_copy(uid_hbm.at[pl.ds(base, BPW)], uid_v)
    pltpu.sync_copy(iid_hbm.at[pl.ds(base, BPW)], iid_v)

    rows16 = lax.iota(jnp.int32, 16)

    def fire(u, v, b):
        ut = pl.multiple_of((u >> 7) * LANES, LANES)
        pltpu.async_copy(utabT_hbm.at[:, pl.ds(ut, LANES)], uslab.at[b],
                         usem.at[b])
        it = pl.multiple_of((v >> 7) * LANES, LANES)
        pltpu.async_copy(itabT_hbm.at[:, pl.ds(it, LANES)], islab.at[b],
                         isem.at[b])

    def extract(u, v, col_i, b):
        pltpu.make_async_copy(utabT_hbm.at[:, pl.ds(0, LANES)], uslab.at[b],
                              usem.at[b]).wait()
        pltpu.make_async_copy(itabT_hbm.at[:, pl.ds(0, LANES)], islab.at[b],
                              isem.at[b]).wait()
        ul = jnp.full((16,), u & (LANES - 1), jnp.int32)
        il = jnp.full((16,), v & (LANES - 1), jnp.int32)
        col = jnp.full((16,), col_i, jnp.int32)
        for c in range(ED // 16):
            r = rows16 + (16 * c)
            uvec = plsc.load_gather(uslab.at[b], [r, ul])
            plsc.store_scatter(uout_v, [r, col], uvec)
            ivec = plsc.load_gather(islab.at[b], [r, il])
            plsc.store_scatter(iout_v, [r, col], ivec)

    for h in range(BPW // HALF):
        h0 = h * HALF

        @pl.loop(h0, h0 + HALF, step=CHUNK)
        def _chunk(o):
            uvec = uid_v[pl.ds(o, CHUNK)]
            ivec = iid_v[pl.ds(o, CHUNK)]

            for j in range(CHUNK):
                i = o + j
                b = j % NBUF
                pj = (j - NBUF) % CHUNK

                @pl.when(i - h0 >= NBUF)
                def _():
                    po = o if j >= NBUF else o - CHUNK
                    puvec = uid_v[pl.ds(po, CHUNK)]
                    pivec = iid_v[pl.ds(po, CHUNK)]
                    extract(puvec[pj], pivec[pj], (po + pj) - h0, b)

                fire(uvec[j], ivec[j], b)

        last = h0 + HALF - CHUNK
        luvec = uid_v[pl.ds(last, CHUNK)]
        livec = iid_v[pl.ds(last, CHUNK)]
        for j in range(NBUF):
            pj = CHUNK - NBUF + j
            extract(luvec[pj], livec[pj], HALF - NBUF + j, pj % NBUF)

        pltpu.sync_copy(uout_v, uoutT_hbm.at[:, pl.ds(base + h0, HALF)])
        pltpu.sync_copy(iout_v, ioutT_hbm.at[:, pl.ds(base + h0, HALF)])


BLK = 2048


def _mlp_body(featT_ref, ueT_ref, ieT_ref, w1t_ref, b1_ref, w2t_ref, b2_ref,
              w3ut_ref, w3it_ref, w3ft_ref, b3_ref, w4t_ref, b4_ref,
              w5t_ref, b5_ref, out_ref):
    dot = lambda a, b: jnp.dot(a, b, preferred_element_type=jnp.float32)
    h = jnp.maximum(dot(w1t_ref[...], featT_ref[...]) + b1_ref[...], 0.0)
    f = jnp.maximum(dot(w2t_ref[...], h) + b2_ref[...], 0.0)
    y = (dot(w3ut_ref[...], ueT_ref[...])
         + dot(w3it_ref[...], ieT_ref[...])
         + dot(w3ft_ref[...], f)
         + b3_ref[...])
    y = jnp.maximum(y, 0.0)
    y = jnp.maximum(dot(w4t_ref[...], y) + b4_ref[...], 0.0)
    z = dot(w5t_ref[...], y) + b5_ref[...]
    out_ref[...] = 1.0 / (1.0 + jnp.exp(-z))


def _mlp_tc(n, featT, ueT, ieT, W1T, b1, W2T, b2, W3uT, W3iT, W3fT, b3,
            W4T, b4, W5T, b5):
    nblk = n // BLK
    col_spec = lambda h: pl.BlockSpec((h, BLK), lambda i: (0, i))
    full = lambda a: pl.BlockSpec(a.shape, lambda i: (0,) * a.ndim)
    return pl.pallas_call(
        _mlp_body,
        grid=(nblk,),
        in_specs=[
            col_spec(featT.shape[0]),
            col_spec(ED),
            col_spec(ED),
            full(W1T), full(b1), full(W2T), full(b2),
            full(W3uT), full(W3iT), full(W3fT), full(b3),
            full(W4T), full(b4), full(W5T), full(b5),
        ],
        out_specs=pl.BlockSpec((1, BLK), lambda i: (0, i)),
        out_shape=jax.ShapeDtypeStruct((1, n), jnp.float32),
    )(featT, ueT, ieT, W1T, b1, W2T, b2, W3uT, W3iT, W3fT, b3,
      W4T, b4, W5T, b5)


def kernel(user_ids, item_ids, features, user_table, item_table,
           W1, b1, W2, b2, W3, b3, W4, b4, W5, b5):
    uid = user_ids.astype(jnp.int32)
    iid = item_ids.astype(jnp.int32)
    utabT = user_table.T
    itabT = item_table.T
    featT = features.T
    ueT_sc, ieT_sc = _gather_sc(uid, iid, utabT, itabT)
    weights = (W1.T, b1.reshape(-1, 1), W2.T, b2.reshape(-1, 1),
               W3[:ED].T, W3[ED:2 * ED].T, W3[2 * ED:].T, b3.reshape(-1, 1),
               W4.T, b4.reshape(-1, 1), W5.T, b5.reshape(-1, 1))
    out = _mlp_tc(BSC, featT, ueT_sc, ieT_sc, *weights)
    return out.reshape(B)
